# baseline (speedup 1.0000x reference)
import functools

import jax
import jax.numpy as jnp
from jax import lax
from jax.experimental import pallas as pl
from jax.experimental.pallas import tpu as pltpu
from jax.experimental.pallas import tpu_sc as plsc

_LANES = 16
_EMBED = 32
_VOCAB = 150


@functools.lru_cache(maxsize=None)
def _build_sc(B, H):
  info = plsc.get_sparse_core_info()
  NC, NS = info.num_cores, info.num_subcores
  NW = NC * NS
  bw = B // NW
  G = bw // _LANES
  TABN = _VOCAB * _EMBED
  F = 2 * _EMBED

  mesh = plsc.VectorSubcoreMesh(core_axis_name="c", subcore_axis_name="s")

  @functools.partial(
      pl.kernel,
      mesh=mesh,
      out_type=jax.ShapeDtypeStruct((B * F,), jnp.float32),
      compiler_params=pltpu.CompilerParams(needs_layout_passes=False),
      scratch_types=[
          pltpu.VMEM((bw * H,), jnp.int32),
          pltpu.VMEM((bw * H,), jnp.int32),
          pltpu.VMEM((TABN,), jnp.float32),
          pltpu.VMEM((TABN,), jnp.float32),
          pltpu.VMEM((bw * F,), jnp.float32),
      ],
  )
  def sck(tr_hbm, td_hbm, ir_hbm, id_hbm, out_hbm, ir_v, id_v, tr_v, td_v, out_v):
    wid = lax.axis_index("s") * NC + lax.axis_index("c")
    base = wid * bw
    pltpu.sync_copy(ir_hbm.at[pl.ds(base * H, bw * H)], ir_v)
    pltpu.sync_copy(id_hbm.at[pl.ds(base * H, bw * H)], id_v)
    pltpu.sync_copy(tr_hbm, tr_v)
    pltpu.sync_copy(td_hbm, td_v)

    iotaH = lax.iota(jnp.int32, _LANES) * H
    iotaF = lax.iota(jnp.int32, _LANES) * F

    def group(g, carry):
      off = g * (_LANES * H)
      idr = []
      idd = []
      for idxv, dst in ((ir_v, idr), (id_v, idd)):
        for h in range(H):
          pos = iotaH + (off + h)
          ids = plsc.load_gather(idxv, [pos])
          dst.append(ids * _EMBED)
      ebase = iotaF + g * (_LANES * F)
      for k in range(_EMBED):
        accr = plsc.load_gather(tr_v, [idr[0] + k])
        for iv in idr[1:]:
          accr = accr + plsc.load_gather(tr_v, [iv + k])
        accd = plsc.load_gather(td_v, [idd[0] + k])
        for iv in idd[1:]:
          accd = accd + plsc.load_gather(td_v, [iv + k])
        plsc.store_scatter(out_v, [ebase + k], accr)
        plsc.store_scatter(out_v, [ebase + (k + _EMBED)], accd)
      return carry

    lax.fori_loop(0, G, group, 0)
    pltpu.sync_copy(out_v, out_hbm.at[pl.ds(base * F, bw * F)])

  return sck


@functools.lru_cache(maxsize=None)
def _build_mlp(B):
  BM = 2048

  def body(x_ref, w1_ref, b1_ref, w2_ref, b2_ref, w3_ref, b3_ref, o_ref):
    h = jnp.maximum(jnp.dot(x_ref[...], w1_ref[...]) + b1_ref[...], 0.0)
    h = jnp.maximum(jnp.dot(h, w2_ref[...]) + b2_ref[...], 0.0)
    o_ref[...] = jnp.maximum(jnp.dot(h, w3_ref[...]) + b3_ref[...], 0.0)

  full = lambda s: pl.BlockSpec(s, lambda i: (0,) * len(s))
  return pl.pallas_call(
      body,
      grid=(B // BM,),
      in_specs=[
          pl.BlockSpec((BM, 2 * _EMBED), lambda i: (i, 0)),
          full((2 * _EMBED, _EMBED)),
          full((_EMBED,)),
          full((_EMBED, _EMBED // 2)),
          full((_EMBED // 2,)),
          full((_EMBED // 2, 1)),
          full((1,)),
      ],
      out_specs=pl.BlockSpec((BM, 1), lambda i: (i, 0)),
      out_shape=jax.ShapeDtypeStruct((B, 1), jnp.float32),
  )


def kernel(radiant_heros, dire_heros, E_r, E_d, W1, b1, W2, b2, W3, b3):
  B, H = radiant_heros.shape
  sck = _build_sc(B, H)
  pooled = sck(E_r.reshape(-1), E_d.reshape(-1),
               radiant_heros.reshape(-1), dire_heros.reshape(-1))
  x = pooled.reshape(B, 2 * _EMBED)
  return _build_mlp(B)(x, W1, b1, W2, b2, W3, b3)


# speedup vs baseline: 1.0821x; 1.0821x over previous
import functools

import jax
import jax.numpy as jnp
from jax import lax
from jax.experimental import pallas as pl
from jax.experimental.pallas import tpu as pltpu
from jax.experimental.pallas import tpu_sc as plsc

_LANES = 16
_EMBED = 32
_VOCAB = 150
_F = 2 * _EMBED


@functools.lru_cache(maxsize=None)
def _sc_info():
  info = plsc.get_sparse_core_info()
  return info.num_cores * info.num_subcores


@functools.lru_cache(maxsize=None)
def _build_sc(B, H, NW):
  NC = plsc.get_sparse_core_info().num_cores
  bw = B // NW
  G = bw // _LANES
  TABN = _VOCAB * _EMBED

  mesh = plsc.VectorSubcoreMesh(core_axis_name="c", subcore_axis_name="s")

  @functools.partial(
      pl.kernel,
      mesh=mesh,
      out_type=jax.ShapeDtypeStruct((B * _F,), jnp.float32),
      compiler_params=pltpu.CompilerParams(needs_layout_passes=False),
      scratch_types=[
          pltpu.VMEM((bw * H,), jnp.int32),
          pltpu.VMEM((bw * H,), jnp.int32),
          pltpu.VMEM((TABN,), jnp.float32),
          pltpu.VMEM((TABN,), jnp.float32),
          pltpu.VMEM((bw * _F,), jnp.float32),
          pltpu.SemaphoreType.DMA,
      ],
  )
  def sck(tr_hbm, td_hbm, ir_hbm, id_hbm, out_hbm, ir_v, id_v, tr_v, td_v, out_v, sem):
    wid = lax.axis_index("s") * NC + lax.axis_index("c")
    base = wid * bw
    c1 = pltpu.async_copy(ir_hbm.at[pl.ds(base * H, bw * H)], ir_v, sem)
    c2 = pltpu.async_copy(id_hbm.at[pl.ds(base * H, bw * H)], id_v, sem)
    c3 = pltpu.async_copy(tr_hbm, tr_v, sem)
    c4 = pltpu.async_copy(td_hbm, td_v, sem)
    c1.wait()
    c2.wait()
    c3.wait()
    c4.wait()

    iotaH = lax.iota(jnp.int32, _LANES) * H

    def group(g, carry):
      off = g * (_LANES * H)
      idr = []
      idd = []
      for idxv, dst in ((ir_v, idr), (id_v, idd)):
        for h in range(H):
          pos = iotaH + (off + h)
          ids = plsc.load_gather(idxv, [pos])
          dst.append(ids * _EMBED)
      eoff = g * _LANES
      for k in range(_EMBED):
        accr = plsc.load_gather(tr_v, [idr[0] + k])
        for iv in idr[1:]:
          accr = accr + plsc.load_gather(tr_v, [iv + k])
        accd = plsc.load_gather(td_v, [idd[0] + k])
        for iv in idd[1:]:
          accd = accd + plsc.load_gather(td_v, [iv + k])
        out_v[pl.ds(k * bw + eoff, _LANES)] = accr
        out_v[pl.ds((k + _EMBED) * bw + eoff, _LANES)] = accd
      return carry

    lax.fori_loop(0, G, group, 0)
    pltpu.sync_copy(out_v, out_hbm.at[pl.ds(base * _F, bw * _F)])

  return sck


@functools.lru_cache(maxsize=None)
def _build_mlp(B, NW):
  bw = B // NW

  def body(x_ref, w1_ref, b1_ref, w2_ref, b2_ref, w3_ref, b3_ref, o_ref):
    cdim0 = (((0,), (0,)), ((), ()))
    h = jnp.maximum(lax.dot_general(w1_ref[...], x_ref[...], cdim0) + b1_ref[...], 0.0)
    h = jnp.maximum(lax.dot_general(w2_ref[...], h, cdim0) + b2_ref[...], 0.0)
    o_ref[...] = jnp.maximum(lax.dot_general(w3_ref[...], h, cdim0) + b3_ref[...],
                             0.0)[None]

  full = lambda s: pl.BlockSpec(s, lambda i: (0,) * len(s))
  return pl.pallas_call(
      body,
      grid=(NW,),
      in_specs=[
          pl.BlockSpec((_F, bw), lambda i: (i, 0)),
          full((_F, _EMBED)),
          full((_EMBED, 1)),
          full((_EMBED, _EMBED // 2)),
          full((_EMBED // 2, 1)),
          full((_EMBED // 2, 1)),
          full((1, 1)),
      ],
      out_specs=pl.BlockSpec((1, 1, bw), lambda i: (i, 0, 0)),
      out_shape=jax.ShapeDtypeStruct((NW, 1, bw), jnp.float32),
  )


def kernel(radiant_heros, dire_heros, E_r, E_d, W1, b1, W2, b2, W3, b3):
  B, H = radiant_heros.shape
  NW = _sc_info()
  sck = _build_sc(B, H, NW)
  pooled = sck(E_r.reshape(-1), E_d.reshape(-1),
               radiant_heros.reshape(-1), dire_heros.reshape(-1))
  x = pooled.reshape(NW * _F, B // NW)
  out = _build_mlp(B, NW)(x, W1, b1.reshape(-1, 1), W2, b2.reshape(-1, 1),
                          W3, b3.reshape(-1, 1))
  return out.reshape(B, 1)


# speedup vs baseline: 1.1710x; 1.0822x over previous
import functools

import jax
import jax.numpy as jnp
from jax import lax
from jax.experimental import pallas as pl
from jax.experimental.pallas import tpu as pltpu
from jax.experimental.pallas import tpu_sc as plsc

_LANES = 16
_EMBED = 32
_VOCAB = 150
_F = 2 * _EMBED


@functools.lru_cache(maxsize=None)
def _sc_info():
  info = plsc.get_sparse_core_info()
  return info.num_cores * info.num_subcores


@functools.lru_cache(maxsize=None)
def _build_sc(B, H, NW):
  NC = plsc.get_sparse_core_info().num_cores
  bw = B // NW
  G = bw // _LANES
  TABN = _VOCAB * _EMBED

  mesh = plsc.VectorSubcoreMesh(core_axis_name="c", subcore_axis_name="s")

  @functools.partial(
      pl.kernel,
      mesh=mesh,
      out_type=jax.ShapeDtypeStruct((B * _F,), jnp.float32),
      compiler_params=pltpu.CompilerParams(needs_layout_passes=False),
      scratch_types=[
          pltpu.VMEM((bw * H,), jnp.int32),
          pltpu.VMEM((bw * H,), jnp.int32),
          pltpu.VMEM((TABN,), jnp.float32),
          pltpu.VMEM((TABN,), jnp.float32),
          pltpu.VMEM((bw * _F,), jnp.float32),
          pltpu.SemaphoreType.DMA,
      ],
  )
  def sck(tr_hbm, td_hbm, ir_hbm, id_hbm, out_hbm, ir_v, id_v, tr_v, td_v, out_v, sem):
    wid = lax.axis_index("s") * NC + lax.axis_index("c")
    base = wid * bw
    c1 = pltpu.async_copy(ir_hbm.at[pl.ds(base * H, bw * H)], ir_v, sem)
    c2 = pltpu.async_copy(id_hbm.at[pl.ds(base * H, bw * H)], id_v, sem)
    c3 = pltpu.async_copy(tr_hbm, tr_v, sem)
    c4 = pltpu.async_copy(td_hbm, td_v, sem)
    c1.wait()
    c2.wait()
    c3.wait()
    c4.wait()

    iotaH = lax.iota(jnp.int32, _LANES) * H

    @plsc.parallel_loop(0, G, 1, unroll=2)
    def group(g):
      off = g * (_LANES * H)
      idr = []
      idd = []
      for idxv, dst in ((ir_v, idr), (id_v, idd)):
        for h in range(H):
          pos = iotaH + (off + h)
          ids = plsc.load_gather(idxv, [pos])
          dst.append(ids * _EMBED)
      eoff = g * _LANES
      for k in range(_EMBED):
        accr = plsc.load_gather(tr_v, [idr[0] + k])
        for iv in idr[1:]:
          accr = accr + plsc.load_gather(tr_v, [iv + k])
        accd = plsc.load_gather(td_v, [idd[0] + k])
        for iv in idd[1:]:
          accd = accd + plsc.load_gather(td_v, [iv + k])
        out_v[pl.ds(k * bw + eoff, _LANES)] = accr
        out_v[pl.ds((k + _EMBED) * bw + eoff, _LANES)] = accd

    pltpu.sync_copy(out_v, out_hbm.at[pl.ds(base * _F, bw * _F)])

  return sck


@functools.lru_cache(maxsize=None)
def _build_mlp(B, NW):
  bw = B // NW

  def body(x_ref, w1_ref, b1_ref, w2_ref, b2_ref, w3_ref, b3_ref, o_ref):
    cdim0 = (((0,), (0,)), ((), ()))
    w1, w2, w3 = w1_ref[...], w2_ref[...], w3_ref[...]
    b1, b2, b3 = b1_ref[...], b2_ref[...], b3_ref[...]
    rows = []
    for w in range(NW):
      x = x_ref[pl.ds(w * _F, _F), :]
      h = jnp.maximum(lax.dot_general(w1, x, cdim0) + b1, 0.0)
      h = jnp.maximum(lax.dot_general(w2, h, cdim0) + b2, 0.0)
      rows.append(jnp.maximum(lax.dot_general(w3, h, cdim0) + b3, 0.0))
    o_ref[...] = jnp.concatenate(rows, axis=0)

  return pl.pallas_call(
      body,
      out_shape=jax.ShapeDtypeStruct((NW, bw), jnp.float32),
  )


def kernel(radiant_heros, dire_heros, E_r, E_d, W1, b1, W2, b2, W3, b3):
  B, H = radiant_heros.shape
  NW = _sc_info()
  sck = _build_sc(B, H, NW)
  pooled = sck(E_r.reshape(-1), E_d.reshape(-1),
               radiant_heros.reshape(-1), dire_heros.reshape(-1))
  x = pooled.reshape(NW * _F, B // NW)
  out = _build_mlp(B, NW)(x, W1, b1.reshape(-1, 1), W2, b2.reshape(-1, 1),
                          W3, b3.reshape(-1, 1))
  return out.reshape(B, 1)


# speedup vs baseline: 1.1763x; 1.0045x over previous
import functools

import jax
import jax.numpy as jnp
from jax import lax
from jax.experimental import pallas as pl
from jax.experimental.pallas import tpu as pltpu
from jax.experimental.pallas import tpu_sc as plsc

_LANES = 16
_EMBED = 32
_VOCAB = 150
_F = 2 * _EMBED


@functools.lru_cache(maxsize=None)
def _sc_info():
  info = plsc.get_sparse_core_info()
  return info.num_cores * info.num_subcores


@functools.lru_cache(maxsize=None)
def _build_sc(B, H, NW):
  NC = plsc.get_sparse_core_info().num_cores
  bw = B // NW
  G = bw // _LANES
  TABN = _VOCAB * _EMBED

  mesh = plsc.VectorSubcoreMesh(core_axis_name="c", subcore_axis_name="s")

  @functools.partial(
      pl.kernel,
      mesh=mesh,
      out_type=jax.ShapeDtypeStruct((B * _F,), jnp.float32),
      compiler_params=pltpu.CompilerParams(needs_layout_passes=False),
      scratch_types=[
          pltpu.VMEM((bw * H,), jnp.int32),
          pltpu.VMEM((bw * H,), jnp.int32),
          pltpu.VMEM((TABN,), jnp.float32),
          pltpu.VMEM((TABN,), jnp.float32),
          pltpu.VMEM((bw * _F,), jnp.float32),
          pltpu.SemaphoreType.DMA,
      ],
  )
  def sck(tr_hbm, td_hbm, ir_hbm, id_hbm, out_hbm, ir_v, id_v, tr_v, td_v, out_v, sem):
    wid = lax.axis_index("s") * NC + lax.axis_index("c")
    base = wid * bw
    c1 = pltpu.async_copy(ir_hbm.at[pl.ds(base * H, bw * H)], ir_v, sem)
    c2 = pltpu.async_copy(id_hbm.at[pl.ds(base * H, bw * H)], id_v, sem)
    c3 = pltpu.async_copy(tr_hbm, tr_v, sem)
    c4 = pltpu.async_copy(td_hbm, td_v, sem)
    c1.wait()
    c2.wait()
    c3.wait()
    c4.wait()

    iotaH = lax.iota(jnp.int32, _LANES) * H

    @plsc.parallel_loop(0, G, 1, unroll=4)
    def group(g):
      off = g * (_LANES * H)
      idr = []
      idd = []
      for idxv, dst in ((ir_v, idr), (id_v, idd)):
        for h in range(H):
          pos = iotaH + (off + h)
          ids = plsc.load_gather(idxv, [pos])
          dst.append(ids * _EMBED)
      eoff = g * _LANES
      for k in range(_EMBED):
        accr = plsc.load_gather(tr_v, [idr[0] + k])
        for iv in idr[1:]:
          accr = accr + plsc.load_gather(tr_v, [iv + k])
        accd = plsc.load_gather(td_v, [idd[0] + k])
        for iv in idd[1:]:
          accd = accd + plsc.load_gather(td_v, [iv + k])
        out_v[pl.ds(k * bw + eoff, _LANES)] = accr
        out_v[pl.ds((k + _EMBED) * bw + eoff, _LANES)] = accd

    pltpu.sync_copy(out_v, out_hbm.at[pl.ds(base * _F, bw * _F)])

  return sck


@functools.lru_cache(maxsize=None)
def _build_mlp(B, NW):
  bw = B // NW

  def body(x_ref, w1_ref, b1_ref, w2_ref, b2_ref, w3_ref, b3_ref, o_ref):
    cdim0 = (((0,), (0,)), ((), ()))
    w1, w2, w3 = w1_ref[...], w2_ref[...], w3_ref[...]
    b1, b2, b3 = b1_ref[...], b2_ref[...], b3_ref[...]
    rows = []
    for w in range(NW):
      x = x_ref[pl.ds(w * _F, _F), :]
      h = jnp.maximum(lax.dot_general(w1, x, cdim0) + b1, 0.0)
      h = jnp.maximum(lax.dot_general(w2, h, cdim0) + b2, 0.0)
      rows.append(jnp.maximum(lax.dot_general(w3, h, cdim0) + b3, 0.0))
    o_ref[...] = jnp.concatenate(rows, axis=0)

  return pl.pallas_call(
      body,
      out_shape=jax.ShapeDtypeStruct((NW, bw), jnp.float32),
  )


def kernel(radiant_heros, dire_heros, E_r, E_d, W1, b1, W2, b2, W3, b3):
  B, H = radiant_heros.shape
  NW = _sc_info()
  sck = _build_sc(B, H, NW)
  pooled = sck(E_r.reshape(-1), E_d.reshape(-1),
               radiant_heros.reshape(-1), dire_heros.reshape(-1))
  x = pooled.reshape(NW * _F, B // NW)
  out = _build_mlp(B, NW)(x, W1, b1.reshape(-1, 1), W2, b2.reshape(-1, 1),
                          W3, b3.reshape(-1, 1))
  return out.reshape(B, 1)
